# baseline (speedup 1.0000x reference)
import functools

import jax
import jax.numpy as jnp
from jax import lax
from jax.experimental import pallas as pl
from jax.experimental.pallas import tpu as pltpu
from jax.experimental.pallas import tpu_sc as plsc

_B = 128
_N = 131072
_TOT = _B * _N
_NW = 32
_PER_W = _TOT // _NW
_C = 16384
_CHUNKS = _PER_W // _C
_LANES = 16

_TIME = 0.5
_V_THRESH = 1.0
_REFRAC_SET = _TIME + 2.0


def _spike_body(mem_hbm, imp_hbm, ref_hbm, spk_out, mem_out, ref_out,
                mbuf, ibuf, rbuf, sem0, sem1, sem2):
    wid = lax.axis_index("s") * 2 + lax.axis_index("c")
    base = wid * _PER_W

    def chunk_body(k, carry):
        off = base + k * _C
        cm = pltpu.async_copy(mem_hbm.at[pl.ds(off, _C)], mbuf, sem0)
        ci = pltpu.async_copy(imp_hbm.at[pl.ds(off, _C)], ibuf, sem1)
        cr = pltpu.async_copy(ref_hbm.at[pl.ds(off, _C)], rbuf, sem2)
        cm.wait()
        ci.wait()
        cr.wait()

        def vec_body(i, c2):
            s = i * _LANES
            m = mbuf[pl.ds(s, _LANES)]
            im = ibuf[pl.ds(s, _LANES)]
            r = rbuf[pl.ds(s, _LANES)]
            nm = m + jnp.where(r > _TIME, 0.0, im)
            cond = nm >= _V_THRESH
            spk = jnp.where(cond, _V_THRESH, 0.0)
            mbuf[pl.ds(s, _LANES)] = nm - spk
            ibuf[pl.ds(s, _LANES)] = spk
            rbuf[pl.ds(s, _LANES)] = jnp.where(cond, _REFRAC_SET, r)
            return c2

        lax.fori_loop(0, _C // _LANES, vec_body, 0)

        cs = pltpu.async_copy(ibuf, spk_out.at[pl.ds(off, _C)], sem0)
        co = pltpu.async_copy(mbuf, mem_out.at[pl.ds(off, _C)], sem1)
        cf = pltpu.async_copy(rbuf, ref_out.at[pl.ds(off, _C)], sem2)
        cs.wait()
        co.wait()
        cf.wait()
        return carry

    lax.fori_loop(0, _CHUNKS, chunk_body, 0)


@jax.jit
def _spike_sc(mem, impulse, refrac_until):
    mesh = plsc.VectorSubcoreMesh(core_axis_name="c", subcore_axis_name="s")
    f = functools.partial(
        pl.kernel,
        out_type=(
            jax.ShapeDtypeStruct((_TOT,), jnp.float32),
            jax.ShapeDtypeStruct((_TOT,), jnp.float32),
            jax.ShapeDtypeStruct((_TOT,), jnp.float32),
        ),
        mesh=mesh,
        scratch_types=[
            pltpu.VMEM((_C,), jnp.float32),
            pltpu.VMEM((_C,), jnp.float32),
            pltpu.VMEM((_C,), jnp.float32),
            pltpu.SemaphoreType.DMA,
            pltpu.SemaphoreType.DMA,
            pltpu.SemaphoreType.DMA,
        ],
    )(_spike_body)
    return f(mem, impulse, refrac_until)


def kernel(mem, impulse, refrac_until):
    spk, nmem, nref = _spike_sc(
        mem.reshape(-1), impulse.reshape(-1), refrac_until.reshape(-1)
    )
    return (spk.reshape(_B, _N), nmem.reshape(_B, _N), nref.reshape(_B, _N))


# speedup vs baseline: 1.5558x; 1.5558x over previous
import functools

import jax
import jax.numpy as jnp
from jax import lax
from jax.experimental import pallas as pl
from jax.experimental.pallas import tpu as pltpu
from jax.experimental.pallas import tpu_sc as plsc

_B = 128
_N = 131072
_TOT = _B * _N
_NW = 32
_PER_W = _TOT // _NW
_C = 8192
_CHUNKS = _PER_W // _C
_CPAIR = _CHUNKS // 2
_LANES = 16

_TIME = 0.5
_V_THRESH = 1.0
_REFRAC_SET = _TIME + 2.0


def _spike_body(mem_hbm, imp_hbm, ref_hbm, spk_out, mem_out, ref_out,
                mi0, ii0, ri0, mi1, ii1, ri1,
                so0, mo0, ro0, so1, mo1, ro1,
                sin0, sin1, sout0, sout1):
    wid = lax.axis_index("s") * 2 + lax.axis_index("c")
    base = wid * _PER_W
    ins = ((mi0, ii0, ri0), (mi1, ii1, ri1))
    outs = ((so0, mo0, ro0), (so1, mo1, ro1))
    sems_in = (sin0, sin1)
    sems_out = (sout0, sout1)

    def start_loads(c, b):
        off = base + c * _C
        pltpu.async_copy(mem_hbm.at[pl.ds(off, _C)], ins[b][0], sems_in[b])
        pltpu.async_copy(imp_hbm.at[pl.ds(off, _C)], ins[b][1], sems_in[b])
        pltpu.async_copy(ref_hbm.at[pl.ds(off, _C)], ins[b][2], sems_in[b])

    def wait_loads(b):
        for buf in ins[b]:
            pltpu.make_async_copy(mem_hbm.at[pl.ds(base, _C)], buf,
                                  sems_in[b]).wait()

    def start_stores(c, b):
        off = base + c * _C
        pltpu.async_copy(outs[b][0], spk_out.at[pl.ds(off, _C)], sems_out[b])
        pltpu.async_copy(outs[b][1], mem_out.at[pl.ds(off, _C)], sems_out[b])
        pltpu.async_copy(outs[b][2], ref_out.at[pl.ds(off, _C)], sems_out[b])

    def wait_stores(b):
        for buf in outs[b]:
            pltpu.make_async_copy(buf, spk_out.at[pl.ds(base, _C)],
                                  sems_out[b]).wait()

    def compute(b):
        mbuf, ibuf, rbuf = ins[b]
        sbuf, obuf, fbuf = outs[b]

        @plsc.parallel_loop(0, _C, step=_LANES, unroll=8)
        def _(s):
            m = mbuf[pl.ds(s, _LANES)]
            im = ibuf[pl.ds(s, _LANES)]
            r = rbuf[pl.ds(s, _LANES)]
            nm = m + jnp.where(r > _TIME, 0.0, im)
            cond = nm >= _V_THRESH
            spk = jnp.where(cond, _V_THRESH, 0.0)
            sbuf[pl.ds(s, _LANES)] = spk
            obuf[pl.ds(s, _LANES)] = nm - spk
            fbuf[pl.ds(s, _LANES)] = jnp.where(cond, _REFRAC_SET, r)

    start_loads(0, 0)
    start_loads(1, 1)
    for b in (0, 1):
        wait_loads(b)
        compute(b)
        start_stores(b, b)
        start_loads(b + 2, b)

    def pair_body(k, carry):
        for b in (0, 1):
            cur = 2 * k + b
            wait_loads(b)
            wait_stores(b)
            compute(b)
            start_stores(cur, b)
            start_loads(cur + 2, b)
        return carry

    lax.fori_loop(1, _CPAIR - 1, pair_body, 0)

    for b in (0, 1):
        wait_loads(b)
        wait_stores(b)
        compute(b)
        start_stores(_CHUNKS - 2 + b, b)
    wait_stores(0)
    wait_stores(1)


@jax.jit
def _spike_sc(mem, impulse, refrac_until):
    mesh = plsc.VectorSubcoreMesh(core_axis_name="c", subcore_axis_name="s")
    f = functools.partial(
        pl.kernel,
        out_type=(
            jax.ShapeDtypeStruct((_TOT,), jnp.float32),
            jax.ShapeDtypeStruct((_TOT,), jnp.float32),
            jax.ShapeDtypeStruct((_TOT,), jnp.float32),
        ),
        mesh=mesh,
        scratch_types=[pltpu.VMEM((_C,), jnp.float32)] * 12
        + [pltpu.SemaphoreType.DMA] * 4,
    )(_spike_body)
    return f(mem, impulse, refrac_until)


def kernel(mem, impulse, refrac_until):
    spk, nmem, nref = _spike_sc(
        mem.reshape(-1), impulse.reshape(-1), refrac_until.reshape(-1)
    )
    return (spk.reshape(_B, _N), nmem.reshape(_B, _N), nref.reshape(_B, _N))


# speedup vs baseline: 4.2904x; 2.7577x over previous
import functools

import jax
import jax.numpy as jnp
from jax import lax
from jax.experimental import pallas as pl
from jax.experimental.pallas import tpu as pltpu
from jax.experimental.pallas import tpu_sc as plsc

_B = 128
_N = 131072
_R = 8
_NW = 32
_HALF = _N // 2
_CC = 1024
_CHUNKS = _HALF // _CC
_CPAIR = _CHUNKS // 2
_LANES = 16

_TIME = 0.5
_V_THRESH = 1.0
_REFRAC_SET = _TIME + 2.0


def _spike_body(mem_hbm, imp_hbm, ref_hbm, spk_out, mem_out, ref_out,
                mi0, ii0, ri0, mi1, ii1, ri1,
                so0, mo0, ro0, so1, mo1, ro1,
                sin0, sin1, sout0, sout1):
    wid = lax.axis_index("s") * 2 + lax.axis_index("c")
    row0 = (wid // 2) * _R
    col0 = (wid % 2) * _HALF
    ins = ((mi0, ii0, ri0), (mi1, ii1, ri1))
    outs = ((so0, mo0, ro0), (so1, mo1, ro1))
    sems_in = (sin0, sin1)
    sems_out = (sout0, sout1)

    def start_loads(c, b):
        cc = col0 + c * _CC
        pltpu.async_copy(mem_hbm.at[pl.ds(row0, _R), pl.ds(cc, _CC)],
                         ins[b][0], sems_in[b])
        pltpu.async_copy(imp_hbm.at[pl.ds(row0, _R), pl.ds(cc, _CC)],
                         ins[b][1], sems_in[b])
        pltpu.async_copy(ref_hbm.at[pl.ds(row0, _R), pl.ds(cc, _CC)],
                         ins[b][2], sems_in[b])

    def wait_loads(b):
        for buf in ins[b]:
            pltpu.make_async_copy(
                mem_hbm.at[pl.ds(row0, _R), pl.ds(col0, _CC)], buf,
                sems_in[b]).wait()

    def start_stores(c, b):
        cc = col0 + c * _CC
        pltpu.async_copy(outs[b][0], spk_out.at[pl.ds(row0, _R), pl.ds(cc, _CC)],
                         sems_out[b])
        pltpu.async_copy(outs[b][1], mem_out.at[pl.ds(row0, _R), pl.ds(cc, _CC)],
                         sems_out[b])
        pltpu.async_copy(outs[b][2], ref_out.at[pl.ds(row0, _R), pl.ds(cc, _CC)],
                         sems_out[b])

    def wait_stores(b):
        for buf in outs[b]:
            pltpu.make_async_copy(
                buf, spk_out.at[pl.ds(row0, _R), pl.ds(col0, _CC)],
                sems_out[b]).wait()

    def compute(b):
        mbuf, ibuf, rbuf = ins[b]
        sbuf, obuf, fbuf = outs[b]

        @plsc.parallel_loop(0, _CC, step=_LANES, unroll=2)
        def _(s):
            for row in range(_R):
                m = mbuf[row, pl.ds(s, _LANES)]
                im = ibuf[row, pl.ds(s, _LANES)]
                r = rbuf[row, pl.ds(s, _LANES)]
                nm = m + jnp.where(r > _TIME, 0.0, im)
                cond = nm >= _V_THRESH
                spk = jnp.where(cond, _V_THRESH, 0.0)
                sbuf[row, pl.ds(s, _LANES)] = spk
                obuf[row, pl.ds(s, _LANES)] = nm - spk
                fbuf[row, pl.ds(s, _LANES)] = jnp.where(cond, _REFRAC_SET, r)

    start_loads(0, 0)
    start_loads(1, 1)
    for b in (0, 1):
        wait_loads(b)
        compute(b)
        start_stores(b, b)
        start_loads(b + 2, b)

    def pair_body(k, carry):
        for b in (0, 1):
            cur = 2 * k + b
            wait_loads(b)
            wait_stores(b)
            compute(b)
            start_stores(cur, b)
            start_loads(cur + 2, b)
        return carry

    lax.fori_loop(1, _CPAIR - 1, pair_body, 0)

    for b in (0, 1):
        wait_loads(b)
        wait_stores(b)
        compute(b)
        start_stores(_CHUNKS - 2 + b, b)
    wait_stores(0)
    wait_stores(1)


@jax.jit
def _spike_sc(mem, impulse, refrac_until):
    mesh = plsc.VectorSubcoreMesh(core_axis_name="c", subcore_axis_name="s")
    f = functools.partial(
        pl.kernel,
        out_type=(
            jax.ShapeDtypeStruct((_B, _N), jnp.float32),
            jax.ShapeDtypeStruct((_B, _N), jnp.float32),
            jax.ShapeDtypeStruct((_B, _N), jnp.float32),
        ),
        mesh=mesh,
        scratch_types=[pltpu.VMEM((_R, _CC), jnp.float32)] * 12
        + [pltpu.SemaphoreType.DMA] * 4,
        compiler_params=pltpu.CompilerParams(use_tc_tiling_on_sc=True),
    )(_spike_body)
    return f(mem, impulse, refrac_until)


def kernel(mem, impulse, refrac_until):
    return _spike_sc(mem, impulse, refrac_until)
